# baseline (speedup 1.0000x reference)
import functools

import jax
import jax.numpy as jnp
from jax import lax
from jax.experimental import pallas as pl
from jax.experimental.pallas import tpu as pltpu

EPS = 1e-5
LANE = 128


def _round_up(x, m):
    return (x + m - 1) // m * m


def _bn_params(st, gb_ref, gi, count):
    tsum = st[0:1, :]
    tsq = st[1:2, :]
    mean = tsum / count
    var = jnp.maximum(tsq / count - mean * mean, 0.0)
    scale = gb_ref[gi:gi + 1, :] * lax.rsqrt(var + EPS)
    shift = gb_ref[gi + 1:gi + 2, :] - mean * scale
    return scale, shift


def _accum_stats(st, n, v):
    ones = jnp.ones((8, v.shape[0]), jnp.bfloat16)
    s1 = jnp.dot(ones, v, preferred_element_type=jnp.float32)[0:1]
    s2 = jnp.dot(ones, v * v, preferred_element_type=jnp.float32)[0:1]
    st[0:1, :] = jnp.where(n == 0, 0.0, st[0:1, :]) + s1
    st[1:2, :] = jnp.where(n == 0, 0.0, st[1:2, :]) + s2


def _bn_relu(v, scale, shift):
    return jnp.maximum(v.astype(jnp.float32) * scale + shift,
                       0.0).astype(jnp.bfloat16)


def _shift_w(a, d, H, W):
    HW, C = a.shape
    z1 = jnp.zeros((1, C), a.dtype)
    if d == -1:
        sh = jnp.concatenate([z1, a[:HW - 1]], axis=0)
        edge = 0
    else:
        sh = jnp.concatenate([a[1:], z1], axis=0)
        edge = W - 1
    col = lax.broadcasted_iota(jnp.int32, (HW, 1), 0) % W
    return jnp.where(col == edge, jnp.zeros_like(sh), sh)


def _shift_h(a, H, W):
    C = a.shape[1]
    return jnp.concatenate([a[W:], jnp.zeros((W, C), a.dtype)], axis=0)


def _conv3x3(a, w3, fpad, H, W, res=None):
    HW = H * W
    C = a.shape[-1]
    fpad[0:W, :] = jnp.zeros((W, 3 * C), jnp.bfloat16)
    fpad[W + HW:2 * W + HW, :] = jnp.zeros((W, 3 * C), jnp.bfloat16)
    fpad[W:W + HW, 0:C] = _shift_w(a, -1, H, W)
    fpad[W:W + HW, C:2 * C] = a
    fpad[W:W + HW, 2 * C:3 * C] = _shift_w(a, 1, H, W)

    acc = jnp.zeros((HW, w3.shape[-1]), jnp.float32)
    for dh in range(3):
        acc = acc + jnp.dot(fpad[dh * W:dh * W + HW, :], w3[dh],
                            preferred_element_type=jnp.float32)
    if res is not None:
        acc = acc + res
    return acc


def _fused_kernel(H, W, count,
                  x_ref, gb_ref, wc_ref, wq_ref,
                  o_ref,
                  xbuf, abuf, bbuf, stx, sth, stx1, sth1, fpad, a5):
    s = pl.program_id(0)
    n = pl.program_id(1)
    HW = H * W
    C = x_ref.shape[1]
    Co = o_ref.shape[1]

    @pl.when(s == 0)
    def _stage_xstats():
        xt = jnp.transpose(x_ref[0], (1, 0))
        xbuf[n] = xt
        _accum_stats(stx, n, xt.astype(jnp.bfloat16))

    @pl.when(s == 1)
    def _stage_conv1():
        scale, shift = _bn_params(stx, gb_ref, 0, count)
        a = _bn_relu(xbuf[n], scale, shift)
        y = _conv3x3(a, wc_ref[0], fpad, H, W).astype(jnp.bfloat16)
        abuf[n] = y
        _accum_stats(sth, n, y)

    @pl.when(s == 2)
    def _stage_conv2():
        scale, shift = _bn_params(sth, gb_ref, 2, count)
        a = _bn_relu(abuf[n], scale, shift)
        y = _conv3x3(a, wc_ref[1], fpad, H, W,
                     res=xbuf[n]).astype(jnp.bfloat16)
        bbuf[n] = y
        _accum_stats(stx1, n, y)

    @pl.when(s == 3)
    def _stage_conv3():
        scale, shift = _bn_params(stx1, gb_ref, 4, count)
        a = _bn_relu(bbuf[n], scale, shift)
        y = _conv3x3(a, wc_ref[2], fpad, H, W).astype(jnp.bfloat16)
        abuf[n] = y
        _accum_stats(sth1, n, y)

    @pl.when(s == 4)
    def _stage_tail():
        s2_, sh2 = _bn_params(sth1, gb_ref, 6, count)
        s3_, sh3 = _bn_params(stx1, gb_ref, 8, count)

        a2 = _bn_relu(abuf[n], s2_, sh2)
        a3 = _bn_relu(bbuf[n], s3_, sh3)

        shp = _shift_w(a2, 1, H, W)
        a5[:, 0:C] = a3
        a5[:, C:2 * C] = a2
        a5[:, 2 * C:3 * C] = shp
        a5[:, 3 * C:4 * C] = _shift_h(a2, H, W)
        a5[:, 4 * C:5 * C] = _shift_h(shp, H, W)

        p00 = jnp.dot(a5[:, 0:2 * C], wq_ref[0:2 * C],
                      preferred_element_type=jnp.float32)
        p01 = jnp.dot(a5[:, C:3 * C], wq_ref[2 * C:4 * C],
                      preferred_element_type=jnp.float32)
        p10 = jnp.dot(a5[:, C:4 * C], wq_ref[4 * C:7 * C],
                      preferred_element_type=jnp.float32)
        p11 = jnp.dot(a5[:, C:5 * C], wq_ref[7 * C:11 * C],
                      preferred_element_type=jnp.float32)

        d0 = jnp.stack([p00, p01], axis=1).reshape(H, 2 * W, Co)
        d1 = jnp.stack([p10, p11], axis=1).reshape(H, 2 * W, Co)
        b = jnp.stack([d0, d1], axis=1).reshape(4 * HW, Co)
        o_ref[0] = jnp.transpose(b, (1, 0))


def _prep_conv_w(w_oihw, cin_p, cout_p):
    k = jnp.transpose(w_oihw.astype(jnp.float32), (2, 3, 1, 0))
    ci, co = k.shape[2], k.shape[3]
    k = k.reshape(9, ci, co)
    k = jnp.pad(k, ((0, 0), (0, cin_p - ci), (0, cout_p - co)))
    return k.reshape(3, 3 * cin_p, cout_p)


def _prep_tail_w(w_iohw, w_sc, cin_p, cout_p):
    k = jnp.transpose(w_iohw.astype(jnp.float32), (2, 3, 0, 1))
    ci, co = k.shape[2], k.shape[3]
    k = k.reshape(9, ci, co)
    k = jnp.pad(k, ((0, 0), (0, cin_p - ci), (0, cout_p - co)))
    sc = jnp.pad(w_sc.astype(jnp.float32),
                 ((0, cin_p - w_sc.shape[0]), (0, cout_p - w_sc.shape[1])))
    z = jnp.zeros_like(sc)

    def tap(kh, kw):
        return k[kh * 3 + kw]

    return jnp.concatenate([
        sc, tap(1, 1),
        tap(1, 2), tap(1, 0),
        tap(2, 1), z, tap(0, 1),
        tap(2, 2), tap(2, 0), tap(0, 2), tap(0, 0),
    ], axis=0).astype(jnp.bfloat16)


def kernel(x, l0_g1, l0_b1, l0_w1, l0_g2, l0_b2, l0_w2,
           l1_g1, l1_b1, l1_w1, l1_g2, l1_b2, l1_w2, l1_g3, l1_b3, l1_w3):
    N, C, H, W = x.shape
    HW = H * W
    Cp = _round_up(C, LANE)
    x0 = x.astype(jnp.float32).reshape(N, C, HW)
    if Cp != C:
        x0 = jnp.pad(x0, ((0, 0), (0, Cp - C), (0, 0)))
    count = float(N * HW)

    Co = l1_w3.shape[1]
    Cop = _round_up(Co, LANE)

    def gbrow(v):
        v = v.astype(jnp.float32)
        if v.shape[0] != Cp:
            v = jnp.pad(v, (0, Cp - v.shape[0]))
        return v.reshape(1, Cp)

    gb = jnp.concatenate([gbrow(v) for v in
                          (l0_g1, l0_b1, l0_g2, l0_b2, l1_g1, l1_b1,
                           l1_g2, l1_b2, l1_g3, l1_b3)], axis=0)
    wc = jnp.stack([_prep_conv_w(l0_w1, Cp, Cp),
                    _prep_conv_w(l0_w2, Cp, Cp),
                    _prep_conv_w(l1_w1, Cp, Cp)]).astype(jnp.bfloat16)
    wq = _prep_tail_w(l1_w2, l1_w3[:, :, 0, 0], Cp, Cop)

    out = pl.pallas_call(
        functools.partial(_fused_kernel, H, W, count),
        out_shape=jax.ShapeDtypeStruct((N, Cop, 4 * HW), jnp.float32),
        grid=(5, N),
        in_specs=[
            pl.BlockSpec((1, Cp, HW),
                         lambda s, n: (jnp.where(s == 0, n, 0), 0, 0)),
            pl.BlockSpec((10, Cp), lambda s, n: (0, 0)),
            pl.BlockSpec((3, 3, 3 * Cp, Cp), lambda s, n: (0, 0, 0, 0)),
            pl.BlockSpec((11 * Cp, Cop), lambda s, n: (0, 0)),
        ],
        out_specs=pl.BlockSpec((1, Cop, 4 * HW),
                               lambda s, n: (jnp.where(s == 4, n, 0), 0, 0)),
        scratch_shapes=[
            pltpu.VMEM((N, HW, Cp), jnp.float32),
            pltpu.VMEM((N, HW, Cp), jnp.bfloat16),
            pltpu.VMEM((N, HW, Cp), jnp.bfloat16),
            pltpu.VMEM((2, Cp), jnp.float32),
            pltpu.VMEM((2, Cp), jnp.float32),
            pltpu.VMEM((2, Cp), jnp.float32),
            pltpu.VMEM((2, Cp), jnp.float32),
            pltpu.VMEM(((H + 2) * W, 3 * Cp), jnp.bfloat16),
            pltpu.VMEM((HW, 5 * Cp), jnp.bfloat16),
        ],
        compiler_params=pltpu.CompilerParams(
            dimension_semantics=("arbitrary", "arbitrary"),
            vmem_limit_bytes=100 * 1024 * 1024),
    )(x0, gb, wc, wq)

    out = out.reshape(N, Cop, 2 * H, 2 * W)
    if Cop != Co:
        out = out[:, :Co]
    return out


# speedup vs baseline: 1.0381x; 1.0381x over previous
import functools

import jax
import jax.numpy as jnp
from jax import lax
from jax.experimental import pallas as pl
from jax.experimental.pallas import tpu as pltpu

EPS = 1e-5
LANE = 128


def _round_up(x, m):
    return (x + m - 1) // m * m


def _bn_params(st, gb_ref, gi, count):
    tsum = st[0:1, :]
    tsq = st[1:2, :]
    mean = tsum / count
    var = jnp.maximum(tsq / count - mean * mean, 0.0)
    scale = gb_ref[gi:gi + 1, :] * lax.rsqrt(var + EPS)
    shift = gb_ref[gi + 1:gi + 2, :] - mean * scale
    return scale, shift


def _accum_stats(st, n, v):
    s1 = jnp.sum(v, axis=0, keepdims=True)
    s2 = jnp.sum(v * v, axis=0, keepdims=True)
    st[0:1, :] = jnp.where(n == 0, 0.0, st[0:1, :]) + s1
    st[1:2, :] = jnp.where(n == 0, 0.0, st[1:2, :]) + s2


def _bn_relu(v, scale, shift):
    return jnp.maximum(v.astype(jnp.float32) * scale + shift,
                       0.0).astype(jnp.bfloat16)


def _shift_w(a, d, H, W):
    HW, C = a.shape
    z1 = jnp.zeros((1, C), a.dtype)
    if d == -1:
        sh = jnp.concatenate([z1, a[:HW - 1]], axis=0)
        edge = 0
    else:
        sh = jnp.concatenate([a[1:], z1], axis=0)
        edge = W - 1
    col = lax.broadcasted_iota(jnp.int32, (HW, 1), 0) % W
    return jnp.where(col == edge, jnp.zeros_like(sh), sh)


def _shift_h(a, H, W):
    C = a.shape[1]
    return jnp.concatenate([a[W:], jnp.zeros((W, C), a.dtype)], axis=0)


def _conv3x3(a, w3, fpad, H, W, res=None):
    HW = H * W
    C = a.shape[-1]
    fpad[0:W, :] = jnp.zeros((W, 3 * C), jnp.bfloat16)
    fpad[W + HW:2 * W + HW, :] = jnp.zeros((W, 3 * C), jnp.bfloat16)
    fpad[W:W + HW, 0:C] = _shift_w(a, -1, H, W)
    fpad[W:W + HW, C:2 * C] = a
    fpad[W:W + HW, 2 * C:3 * C] = _shift_w(a, 1, H, W)

    acc = jnp.zeros((HW, w3.shape[-1]), jnp.float32)
    for dh in range(3):
        acc = acc + jnp.dot(fpad[dh * W:dh * W + HW, :], w3[dh],
                            preferred_element_type=jnp.float32)
    if res is not None:
        acc = acc + res
    return acc


def _fused_kernel(H, W, count,
                  x_ref, gb_ref, wc_ref, wq_ref,
                  o_ref,
                  xbuf, abuf, bbuf, stx, sth, stx1, sth1, fpad, a5):
    s = pl.program_id(0)
    n = pl.program_id(1)
    HW = H * W
    C = x_ref.shape[1]
    Co = o_ref.shape[1]

    @pl.when(s == 0)
    def _stage_xstats():
        xt = jnp.transpose(x_ref[0], (1, 0))
        xbuf[n] = xt
        _accum_stats(stx, n, xt)

    @pl.when(s == 1)
    def _stage_conv1():
        scale, shift = _bn_params(stx, gb_ref, 0, count)
        a = _bn_relu(xbuf[n], scale, shift)
        acc = _conv3x3(a, wc_ref[0], fpad, H, W)
        abuf[n] = acc.astype(jnp.bfloat16)
        _accum_stats(sth, n, acc)

    @pl.when(s == 2)
    def _stage_conv2():
        scale, shift = _bn_params(sth, gb_ref, 2, count)
        a = _bn_relu(abuf[n], scale, shift)
        acc = _conv3x3(a, wc_ref[1], fpad, H, W, res=xbuf[n])
        bbuf[n] = acc.astype(jnp.bfloat16)
        _accum_stats(stx1, n, acc)

    @pl.when(s == 3)
    def _stage_conv3():
        scale, shift = _bn_params(stx1, gb_ref, 4, count)
        a = _bn_relu(bbuf[n], scale, shift)
        acc = _conv3x3(a, wc_ref[2], fpad, H, W)
        abuf[n] = acc.astype(jnp.bfloat16)
        _accum_stats(sth1, n, acc)

    @pl.when(s == 4)
    def _stage_tail():
        s2_, sh2 = _bn_params(sth1, gb_ref, 6, count)
        s3_, sh3 = _bn_params(stx1, gb_ref, 8, count)

        a2 = _bn_relu(abuf[n], s2_, sh2)
        a3 = _bn_relu(bbuf[n], s3_, sh3)

        shp = _shift_w(a2, 1, H, W)
        a5[:, 0:C] = a3
        a5[:, C:2 * C] = a2
        a5[:, 2 * C:3 * C] = shp
        a5[:, 3 * C:4 * C] = _shift_h(a2, H, W)
        a5[:, 4 * C:5 * C] = _shift_h(shp, H, W)

        p00 = jnp.dot(a5[:, 0:2 * C], wq_ref[0:2 * C],
                      preferred_element_type=jnp.float32)
        p01 = jnp.dot(a5[:, C:3 * C], wq_ref[2 * C:4 * C],
                      preferred_element_type=jnp.float32)
        p10 = jnp.dot(a5[:, C:4 * C], wq_ref[4 * C:7 * C],
                      preferred_element_type=jnp.float32)
        p11 = jnp.dot(a5[:, C:5 * C], wq_ref[7 * C:11 * C],
                      preferred_element_type=jnp.float32)

        d0 = jnp.stack([p00, p01], axis=1).reshape(H, 2 * W, Co)
        d1 = jnp.stack([p10, p11], axis=1).reshape(H, 2 * W, Co)
        b = jnp.stack([d0, d1], axis=1).reshape(4 * HW, Co)
        o_ref[0] = jnp.transpose(b, (1, 0))


def _prep_conv_w(w_oihw, cin_p, cout_p):
    k = jnp.transpose(w_oihw.astype(jnp.float32), (2, 3, 1, 0))
    ci, co = k.shape[2], k.shape[3]
    k = k.reshape(9, ci, co)
    k = jnp.pad(k, ((0, 0), (0, cin_p - ci), (0, cout_p - co)))
    return k.reshape(3, 3 * cin_p, cout_p)


def _prep_tail_w(w_iohw, w_sc, cin_p, cout_p):
    k = jnp.transpose(w_iohw.astype(jnp.float32), (2, 3, 0, 1))
    ci, co = k.shape[2], k.shape[3]
    k = k.reshape(9, ci, co)
    k = jnp.pad(k, ((0, 0), (0, cin_p - ci), (0, cout_p - co)))
    sc = jnp.pad(w_sc.astype(jnp.float32),
                 ((0, cin_p - w_sc.shape[0]), (0, cout_p - w_sc.shape[1])))
    z = jnp.zeros_like(sc)

    def tap(kh, kw):
        return k[kh * 3 + kw]

    return jnp.concatenate([
        sc, tap(1, 1),
        tap(1, 2), tap(1, 0),
        tap(2, 1), z, tap(0, 1),
        tap(2, 2), tap(2, 0), tap(0, 2), tap(0, 0),
    ], axis=0).astype(jnp.bfloat16)


def kernel(x, l0_g1, l0_b1, l0_w1, l0_g2, l0_b2, l0_w2,
           l1_g1, l1_b1, l1_w1, l1_g2, l1_b2, l1_w2, l1_g3, l1_b3, l1_w3):
    N, C, H, W = x.shape
    HW = H * W
    Cp = _round_up(C, LANE)
    x0 = x.astype(jnp.float32).reshape(N, C, HW)
    if Cp != C:
        x0 = jnp.pad(x0, ((0, 0), (0, Cp - C), (0, 0)))
    count = float(N * HW)

    Co = l1_w3.shape[1]
    Cop = _round_up(Co, LANE)

    def gbrow(v):
        v = v.astype(jnp.float32)
        if v.shape[0] != Cp:
            v = jnp.pad(v, (0, Cp - v.shape[0]))
        return v.reshape(1, Cp)

    gb = jnp.concatenate([gbrow(v) for v in
                          (l0_g1, l0_b1, l0_g2, l0_b2, l1_g1, l1_b1,
                           l1_g2, l1_b2, l1_g3, l1_b3)], axis=0)
    wc = jnp.stack([_prep_conv_w(l0_w1, Cp, Cp),
                    _prep_conv_w(l0_w2, Cp, Cp),
                    _prep_conv_w(l1_w1, Cp, Cp)]).astype(jnp.bfloat16)
    wq = _prep_tail_w(l1_w2, l1_w3[:, :, 0, 0], Cp, Cop)

    out = pl.pallas_call(
        functools.partial(_fused_kernel, H, W, count),
        out_shape=jax.ShapeDtypeStruct((N, Cop, 4 * HW), jnp.float32),
        grid=(5, N),
        in_specs=[
            pl.BlockSpec((1, Cp, HW),
                         lambda s, n: (jnp.where(s == 0, n, 0), 0, 0)),
            pl.BlockSpec((10, Cp), lambda s, n: (0, 0)),
            pl.BlockSpec((3, 3, 3 * Cp, Cp), lambda s, n: (0, 0, 0, 0)),
            pl.BlockSpec((11 * Cp, Cop), lambda s, n: (0, 0)),
        ],
        out_specs=pl.BlockSpec((1, Cop, 4 * HW),
                               lambda s, n: (jnp.where(s == 4, n, 0), 0, 0)),
        scratch_shapes=[
            pltpu.VMEM((N, HW, Cp), jnp.float32),
            pltpu.VMEM((N, HW, Cp), jnp.bfloat16),
            pltpu.VMEM((N, HW, Cp), jnp.bfloat16),
            pltpu.VMEM((2, Cp), jnp.float32),
            pltpu.VMEM((2, Cp), jnp.float32),
            pltpu.VMEM((2, Cp), jnp.float32),
            pltpu.VMEM((2, Cp), jnp.float32),
            pltpu.VMEM(((H + 2) * W, 3 * Cp), jnp.bfloat16),
            pltpu.VMEM((HW, 5 * Cp), jnp.bfloat16),
        ],
        compiler_params=pltpu.CompilerParams(
            dimension_semantics=("arbitrary", "arbitrary"),
            vmem_limit_bytes=100 * 1024 * 1024),
    )(x0, gb, wc, wq)

    out = out.reshape(N, Cop, 2 * H, 2 * W)
    if Cop != Co:
        out = out[:, :Co]
    return out


# speedup vs baseline: 1.0586x; 1.0198x over previous
import functools

import jax
import jax.numpy as jnp
from jax import lax
from jax.experimental import pallas as pl
from jax.experimental.pallas import tpu as pltpu

EPS = 1e-5
LANE = 128


def _round_up(x, m):
    return (x + m - 1) // m * m


def _bn_params(st, g_ref, b_ref, count):
    tsum = st[0:1, :]
    tsq = st[1:2, :]
    mean = tsum / count
    var = jnp.maximum(tsq / count - mean * mean, 0.0)
    scale = g_ref[...].astype(jnp.float32) * lax.rsqrt(var + EPS)
    shift = b_ref[...].astype(jnp.float32) - mean * scale
    return scale, shift


def _accum_stats(st, n, v):
    s1 = jnp.sum(v, axis=0, keepdims=True)
    s2 = jnp.sum(v * v, axis=0, keepdims=True)
    st[0:1, :] = jnp.where(n == 0, 0.0, st[0:1, :]) + s1
    st[1:2, :] = jnp.where(n == 0, 0.0, st[1:2, :]) + s2


def _bn_relu(v, scale, shift):
    return jnp.maximum(v.astype(jnp.float32) * scale + shift,
                       0.0).astype(jnp.bfloat16)


def _shift_w(a, d, H, W):
    HW, C = a.shape
    z1 = jnp.zeros((1, C), a.dtype)
    if d == -1:
        sh = jnp.concatenate([z1, a[:HW - 1]], axis=0)
        edge = 0
    else:
        sh = jnp.concatenate([a[1:], z1], axis=0)
        edge = W - 1
    col = lax.broadcasted_iota(jnp.int32, (HW, 1), 0) % W
    return jnp.where(col == edge, jnp.zeros_like(sh), sh)


def _shift_h(a, H, W):
    C = a.shape[1]
    return jnp.concatenate([a[W:], jnp.zeros((W, C), a.dtype)], axis=0)


def _conv3x3(a, w3, fpad, H, W, res=None):
    HW = H * W
    C = a.shape[-1]
    fpad[0:W, :] = jnp.zeros((W, 3 * C), jnp.bfloat16)
    fpad[W + HW:2 * W + HW, :] = jnp.zeros((W, 3 * C), jnp.bfloat16)
    fpad[W:W + HW, 0:C] = _shift_w(a, -1, H, W)
    fpad[W:W + HW, C:2 * C] = a
    fpad[W:W + HW, 2 * C:3 * C] = _shift_w(a, 1, H, W)

    acc = jnp.zeros((HW, w3.shape[-1]), jnp.float32)
    for dh in range(3):
        acc = acc + jnp.dot(fpad[dh * W:dh * W + HW, :], w3[dh],
                            preferred_element_type=jnp.float32)
    if res is not None:
        acc = acc + res
    return acc


def _fused_kernel(H, W, count,
                  x_ref, g1_ref, b1_ref, w1_ref, g2_ref, b2_ref, w2_ref,
                  g3_ref, b3_ref, w3_ref, g4_ref, b4_ref,
                  wqa_ref, wqb_ref, wqc_ref, wqd_ref,
                  g5_ref, b5_ref,
                  o_ref,
                  xbuf, abuf, bbuf, stx, sth, stx1, sth1, fpad, a5):
    s = pl.program_id(0)
    n = pl.program_id(1)
    HW = H * W
    C = x_ref.shape[1]
    Co = o_ref.shape[1]

    @pl.when(s == 0)
    def _stage_xstats():
        xt = jnp.transpose(x_ref[0], (1, 0))
        xbuf[n] = xt
        _accum_stats(stx, n, xt)

    @pl.when(s == 1)
    def _stage_conv1():
        scale, shift = _bn_params(stx, g1_ref, b1_ref, count)
        a = _bn_relu(xbuf[n], scale, shift)
        acc = _conv3x3(a, w1_ref, fpad, H, W)
        abuf[n] = acc.astype(jnp.bfloat16)
        _accum_stats(sth, n, acc)

    @pl.when(s == 2)
    def _stage_conv2():
        scale, shift = _bn_params(sth, g2_ref, b2_ref, count)
        a = _bn_relu(abuf[n], scale, shift)
        acc = _conv3x3(a, w2_ref, fpad, H, W, res=xbuf[n])
        bbuf[n] = acc.astype(jnp.bfloat16)
        _accum_stats(stx1, n, acc)

    @pl.when(s == 3)
    def _stage_conv3():
        scale, shift = _bn_params(stx1, g3_ref, b3_ref, count)
        a = _bn_relu(bbuf[n], scale, shift)
        acc = _conv3x3(a, w3_ref, fpad, H, W)
        abuf[n] = acc.astype(jnp.bfloat16)
        _accum_stats(sth1, n, acc)

    @pl.when(s == 4)
    def _stage_tail():
        s2_, sh2 = _bn_params(sth1, g4_ref, b4_ref, count)
        s3_, sh3 = _bn_params(stx1, g5_ref, b5_ref, count)

        a2 = _bn_relu(abuf[n], s2_, sh2)
        a3 = _bn_relu(bbuf[n], s3_, sh3)

        shp = _shift_w(a2, 1, H, W)
        a5[:, 0:C] = a3
        a5[:, C:2 * C] = a2
        a5[:, 2 * C:3 * C] = shp
        a5[:, 3 * C:4 * C] = _shift_h(a2, H, W)
        a5[:, 4 * C:5 * C] = _shift_h(shp, H, W)

        p00 = jnp.dot(a5[:, 0:2 * C], wqa_ref[...],
                      preferred_element_type=jnp.float32)
        p01 = jnp.dot(a5[:, C:3 * C], wqb_ref[...],
                      preferred_element_type=jnp.float32)
        p10 = jnp.dot(a5[:, C:4 * C], wqc_ref[...],
                      preferred_element_type=jnp.float32)
        p11 = jnp.dot(a5[:, C:5 * C], wqd_ref[...],
                      preferred_element_type=jnp.float32)

        d0 = jnp.stack([p00, p01], axis=1).reshape(H, 2 * W, Co)
        d1 = jnp.stack([p10, p11], axis=1).reshape(H, 2 * W, Co)
        b = jnp.stack([d0, d1], axis=1).reshape(4 * HW, Co)
        o_ref[0] = jnp.transpose(b, (1, 0))


def _prep_conv_w(w_oihw, cin_p, cout_p):
    k = jnp.transpose(w_oihw.astype(jnp.float32), (2, 3, 1, 0))
    ci, co = k.shape[2], k.shape[3]
    k = k.reshape(9, ci, co)
    k = jnp.pad(k, ((0, 0), (0, cin_p - ci), (0, cout_p - co)))
    return k.reshape(3, 3 * cin_p, cout_p).astype(jnp.bfloat16)


def _prep_tail_w(w_iohw, w_sc, cin_p, cout_p):
    k = jnp.transpose(w_iohw.astype(jnp.float32), (2, 3, 0, 1))
    ci, co = k.shape[2], k.shape[3]
    k = k.reshape(9, ci, co)
    k = jnp.pad(k, ((0, 0), (0, cin_p - ci), (0, cout_p - co)))
    sc = jnp.pad(w_sc.astype(jnp.float32),
                 ((0, cin_p - w_sc.shape[0]), (0, cout_p - w_sc.shape[1])))
    z = jnp.zeros_like(sc)

    def tap(kh, kw):
        return k[kh * 3 + kw]

    p00 = jnp.concatenate([sc, tap(1, 1)], axis=0)
    p01 = jnp.concatenate([tap(1, 2), tap(1, 0)], axis=0)
    p10 = jnp.concatenate([tap(2, 1), z, tap(0, 1)], axis=0)
    p11 = jnp.concatenate([tap(2, 2), tap(2, 0), tap(0, 2), tap(0, 0)],
                          axis=0)
    return (p00.astype(jnp.bfloat16), p01.astype(jnp.bfloat16),
            p10.astype(jnp.bfloat16), p11.astype(jnp.bfloat16))


def kernel(x, l0_g1, l0_b1, l0_w1, l0_g2, l0_b2, l0_w2,
           l1_g1, l1_b1, l1_w1, l1_g2, l1_b2, l1_w2, l1_g3, l1_b3, l1_w3):
    N, C, H, W = x.shape
    HW = H * W
    Cp = _round_up(C, LANE)
    x0 = x.astype(jnp.float32).reshape(N, C, HW)
    if Cp != C:
        x0 = jnp.pad(x0, ((0, 0), (0, Cp - C), (0, 0)))
    count = float(N * HW)

    Co = l1_w3.shape[1]
    Cop = _round_up(Co, LANE)

    def gbrow(v):
        v = v.astype(jnp.float32)
        if v.shape[0] != Cp:
            v = jnp.pad(v, (0, Cp - v.shape[0]))
        return v.reshape(1, Cp)

    wqa, wqb, wqc, wqd = _prep_tail_w(l1_w2, l1_w3[:, :, 0, 0], Cp, Cop)

    cgrid = pl.BlockSpec((1, Cp), lambda s, n: (0, 0))
    wgrid = pl.BlockSpec((3, 3 * Cp, Cp), lambda s, n: (0, 0, 0))
    out = pl.pallas_call(
        functools.partial(_fused_kernel, H, W, count),
        out_shape=jax.ShapeDtypeStruct((N, Cop, 4 * HW), jnp.float32),
        grid=(5, N),
        in_specs=[
            pl.BlockSpec((1, Cp, HW),
                         lambda s, n: (jnp.where(s == 0, n, 0), 0, 0)),
            cgrid, cgrid, wgrid,
            cgrid, cgrid, wgrid,
            cgrid, cgrid, wgrid,
            cgrid, cgrid,
            pl.BlockSpec((2 * Cp, Cop), lambda s, n: (0, 0)),
            pl.BlockSpec((2 * Cp, Cop), lambda s, n: (0, 0)),
            pl.BlockSpec((3 * Cp, Cop), lambda s, n: (0, 0)),
            pl.BlockSpec((4 * Cp, Cop), lambda s, n: (0, 0)),
            cgrid, cgrid,
        ],
        out_specs=pl.BlockSpec((1, Cop, 4 * HW),
                               lambda s, n: (jnp.where(s == 4, n, 0), 0, 0)),
        scratch_shapes=[
            pltpu.VMEM((N, HW, Cp), jnp.float32),
            pltpu.VMEM((N, HW, Cp), jnp.bfloat16),
            pltpu.VMEM((N, HW, Cp), jnp.bfloat16),
            pltpu.VMEM((2, Cp), jnp.float32),
            pltpu.VMEM((2, Cp), jnp.float32),
            pltpu.VMEM((2, Cp), jnp.float32),
            pltpu.VMEM((2, Cp), jnp.float32),
            pltpu.VMEM(((H + 2) * W, 3 * Cp), jnp.bfloat16),
            pltpu.VMEM((HW, 5 * Cp), jnp.bfloat16),
        ],
        compiler_params=pltpu.CompilerParams(
            dimension_semantics=("arbitrary", "arbitrary"),
            vmem_limit_bytes=100 * 1024 * 1024),
    )(x0, gbrow(l0_g1), gbrow(l0_b1), _prep_conv_w(l0_w1, Cp, Cp),
      gbrow(l0_g2), gbrow(l0_b2), _prep_conv_w(l0_w2, Cp, Cp),
      gbrow(l1_g1), gbrow(l1_b1), _prep_conv_w(l1_w1, Cp, Cp),
      gbrow(l1_g2), gbrow(l1_b2), wqa, wqb, wqc, wqd,
      gbrow(l1_g3), gbrow(l1_b3))

    out = out.reshape(N, Cop, 2 * H, 2 * W)
    if Cop != Co:
        out = out[:, :Co]
    return out


# speedup vs baseline: 1.0883x; 1.0280x over previous
import functools

import jax
import jax.numpy as jnp
from jax import lax
from jax.experimental import pallas as pl
from jax.experimental.pallas import tpu as pltpu

EPS = 1e-5
LANE = 128


def _round_up(x, m):
    return (x + m - 1) // m * m


def _bn_params(st, g_ref, b_ref, count):
    tsum = st[0:1, :]
    tsq = st[1:2, :]
    mean = tsum / count
    var = jnp.maximum(tsq / count - mean * mean, 0.0)
    scale = g_ref[...].astype(jnp.float32) * lax.rsqrt(var + EPS)
    shift = b_ref[...].astype(jnp.float32) - mean * scale
    return scale, shift


def _accum_stats(st, n, v):
    s1 = jnp.sum(v, axis=0, keepdims=True)
    s2 = jnp.sum(v * v, axis=0, keepdims=True)
    st[0:1, :] = jnp.where(n == 0, 0.0, st[0:1, :]) + s1
    st[1:2, :] = jnp.where(n == 0, 0.0, st[1:2, :]) + s2


def _bn_relu(v, scale, shift):
    return jnp.maximum(v.astype(jnp.float32) * scale + shift,
                       0.0).astype(jnp.bfloat16)


def _shift_w(a, d, H, W):
    HW, C = a.shape
    z1 = jnp.zeros((1, C), a.dtype)
    if d == -1:
        sh = jnp.concatenate([z1, a[:HW - 1]], axis=0)
        edge = 0
    else:
        sh = jnp.concatenate([a[1:], z1], axis=0)
        edge = W - 1
    col = lax.broadcasted_iota(jnp.int32, (HW, 1), 0) % W
    return jnp.where(col == edge, jnp.zeros_like(sh), sh)


def _shift_h(a, H, W):
    C = a.shape[1]
    return jnp.concatenate([a[W:], jnp.zeros((W, C), a.dtype)], axis=0)


def _conv3x3_pair(a0, a1, w3, fpad, H, W):
    HW = H * W
    C = a0.shape[-1]
    R = (H + 2) * W
    for g, a in ((0, a0), (1, a1)):
        o = g * R
        fpad[o:o + W, :] = jnp.zeros((W, 3 * C), jnp.bfloat16)
        fpad[o + W + HW:o + 2 * W + HW, :] = jnp.zeros((W, 3 * C),
                                                       jnp.bfloat16)
        fpad[o + W:o + W + HW, 0:C] = _shift_w(a, -1, H, W)
        fpad[o + W:o + W + HW, C:2 * C] = a
        fpad[o + W:o + W + HW, 2 * C:3 * C] = _shift_w(a, 1, H, W)

    M = R + HW
    acc = jnp.zeros((M, w3.shape[-1]), jnp.float32)
    for dh in range(3):
        acc = acc + jnp.dot(fpad[dh * W:dh * W + M, :], w3[dh],
                            preferred_element_type=jnp.float32)
    return acc[0:HW], acc[R:R + HW]


def _fused_kernel(H, W, count,
                  x_ref, g1_ref, b1_ref, w1_ref, g2_ref, b2_ref, w2_ref,
                  g3_ref, b3_ref, w3_ref, g4_ref, b4_ref,
                  wqa_ref, wqb_ref, wqc_ref, wqd_ref,
                  g5_ref, b5_ref,
                  o_ref,
                  xbuf, abuf, bbuf, stx, sth, stx1, sth1, fpad, a5):
    s = pl.program_id(0)
    n = pl.program_id(1)
    HW = H * W
    C = x_ref.shape[1]
    Co = o_ref.shape[1]

    @pl.when(s == 0)
    def _stage_xstats():
        xt = jnp.transpose(x_ref[0], (1, 0))
        xbuf[n] = xt
        _accum_stats(stx, n, xt)

    pair = jnp.logical_and(n % 2 == 0, True)

    @pl.when(jnp.logical_and(s == 1, pair))
    def _stage_conv1():
        scale, shift = _bn_params(stx, g1_ref, b1_ref, count)
        a0 = _bn_relu(xbuf[n], scale, shift)
        a1 = _bn_relu(xbuf[n + 1], scale, shift)
        acc0, acc1 = _conv3x3_pair(a0, a1, w1_ref, fpad, H, W)
        abuf[n] = acc0.astype(jnp.bfloat16)
        abuf[n + 1] = acc1.astype(jnp.bfloat16)
        _accum_stats(sth, n, acc0)
        _accum_stats(sth, n + 1, acc1)

    @pl.when(jnp.logical_and(s == 2, pair))
    def _stage_conv2():
        scale, shift = _bn_params(sth, g2_ref, b2_ref, count)
        a0 = _bn_relu(abuf[n], scale, shift)
        a1 = _bn_relu(abuf[n + 1], scale, shift)
        acc0, acc1 = _conv3x3_pair(a0, a1, w2_ref, fpad, H, W)
        acc0 = acc0 + xbuf[n]
        acc1 = acc1 + xbuf[n + 1]
        bbuf[n] = acc0.astype(jnp.bfloat16)
        bbuf[n + 1] = acc1.astype(jnp.bfloat16)
        _accum_stats(stx1, n, acc0)
        _accum_stats(stx1, n + 1, acc1)

    @pl.when(jnp.logical_and(s == 3, pair))
    def _stage_conv3():
        scale, shift = _bn_params(stx1, g3_ref, b3_ref, count)
        a0 = _bn_relu(bbuf[n], scale, shift)
        a1 = _bn_relu(bbuf[n + 1], scale, shift)
        acc0, acc1 = _conv3x3_pair(a0, a1, w3_ref, fpad, H, W)
        abuf[n] = acc0.astype(jnp.bfloat16)
        abuf[n + 1] = acc1.astype(jnp.bfloat16)
        _accum_stats(sth1, n, acc0)
        _accum_stats(sth1, n + 1, acc1)

    @pl.when(s == 4)
    def _stage_tail():
        s2_, sh2 = _bn_params(sth1, g4_ref, b4_ref, count)
        s3_, sh3 = _bn_params(stx1, g5_ref, b5_ref, count)

        a2 = _bn_relu(abuf[n], s2_, sh2)
        a3 = _bn_relu(bbuf[n], s3_, sh3)

        shp = _shift_w(a2, 1, H, W)
        a5[:, 0:C] = a3
        a5[:, C:2 * C] = a2
        a5[:, 2 * C:3 * C] = shp
        a5[:, 3 * C:4 * C] = _shift_h(a2, H, W)
        a5[:, 4 * C:5 * C] = _shift_h(shp, H, W)

        p00 = jnp.dot(a5[:, 0:2 * C], wqa_ref[...],
                      preferred_element_type=jnp.float32)
        p01 = jnp.dot(a5[:, C:3 * C], wqb_ref[...],
                      preferred_element_type=jnp.float32)
        p10 = jnp.dot(a5[:, C:4 * C], wqc_ref[...],
                      preferred_element_type=jnp.float32)
        p11 = jnp.dot(a5[:, C:5 * C], wqd_ref[...],
                      preferred_element_type=jnp.float32)

        d0 = jnp.stack([p00, p01], axis=1).reshape(H, 2 * W, Co)
        d1 = jnp.stack([p10, p11], axis=1).reshape(H, 2 * W, Co)
        b = jnp.stack([d0, d1], axis=1).reshape(4 * HW, Co)
        o_ref[0] = jnp.transpose(b, (1, 0))


def _prep_conv_w(w_oihw, cin_p, cout_p):
    k = jnp.transpose(w_oihw.astype(jnp.float32), (2, 3, 1, 0))
    ci, co = k.shape[2], k.shape[3]
    k = k.reshape(9, ci, co)
    k = jnp.pad(k, ((0, 0), (0, cin_p - ci), (0, cout_p - co)))
    return k.reshape(3, 3 * cin_p, cout_p).astype(jnp.bfloat16)


def _prep_tail_w(w_iohw, w_sc, cin_p, cout_p):
    k = jnp.transpose(w_iohw.astype(jnp.float32), (2, 3, 0, 1))
    ci, co = k.shape[2], k.shape[3]
    k = k.reshape(9, ci, co)
    k = jnp.pad(k, ((0, 0), (0, cin_p - ci), (0, cout_p - co)))
    sc = jnp.pad(w_sc.astype(jnp.float32),
                 ((0, cin_p - w_sc.shape[0]), (0, cout_p - w_sc.shape[1])))
    z = jnp.zeros_like(sc)

    def tap(kh, kw):
        return k[kh * 3 + kw]

    p00 = jnp.concatenate([sc, tap(1, 1)], axis=0)
    p01 = jnp.concatenate([tap(1, 2), tap(1, 0)], axis=0)
    p10 = jnp.concatenate([tap(2, 1), z, tap(0, 1)], axis=0)
    p11 = jnp.concatenate([tap(2, 2), tap(2, 0), tap(0, 2), tap(0, 0)],
                          axis=0)
    return (p00.astype(jnp.bfloat16), p01.astype(jnp.bfloat16),
            p10.astype(jnp.bfloat16), p11.astype(jnp.bfloat16))


def kernel(x, l0_g1, l0_b1, l0_w1, l0_g2, l0_b2, l0_w2,
           l1_g1, l1_b1, l1_w1, l1_g2, l1_b2, l1_w2, l1_g3, l1_b3, l1_w3):
    N, C, H, W = x.shape
    HW = H * W
    Cp = _round_up(C, LANE)
    x0 = x.astype(jnp.float32).reshape(N, C, HW)
    if Cp != C:
        x0 = jnp.pad(x0, ((0, 0), (0, Cp - C), (0, 0)))
    count = float(N * HW)

    Co = l1_w3.shape[1]
    Cop = _round_up(Co, LANE)

    def gbrow(v):
        v = v.astype(jnp.float32)
        if v.shape[0] != Cp:
            v = jnp.pad(v, (0, Cp - v.shape[0]))
        return v.reshape(1, Cp)

    wqa, wqb, wqc, wqd = _prep_tail_w(l1_w2, l1_w3[:, :, 0, 0], Cp, Cop)

    cgrid = pl.BlockSpec((1, Cp), lambda s, n: (0, 0))
    wgrid = pl.BlockSpec((3, 3 * Cp, Cp), lambda s, n: (0, 0, 0))
    out = pl.pallas_call(
        functools.partial(_fused_kernel, H, W, count),
        out_shape=jax.ShapeDtypeStruct((N, Cop, 4 * HW), jnp.float32),
        grid=(5, N),
        in_specs=[
            pl.BlockSpec((1, Cp, HW),
                         lambda s, n: (jnp.where(s == 0, n, 0), 0, 0)),
            cgrid, cgrid, wgrid,
            cgrid, cgrid, wgrid,
            cgrid, cgrid, wgrid,
            cgrid, cgrid,
            pl.BlockSpec((2 * Cp, Cop), lambda s, n: (0, 0)),
            pl.BlockSpec((2 * Cp, Cop), lambda s, n: (0, 0)),
            pl.BlockSpec((3 * Cp, Cop), lambda s, n: (0, 0)),
            pl.BlockSpec((4 * Cp, Cop), lambda s, n: (0, 0)),
            cgrid, cgrid,
        ],
        out_specs=pl.BlockSpec((1, Cop, 4 * HW),
                               lambda s, n: (jnp.where(s == 4, n, 0), 0, 0)),
        scratch_shapes=[
            pltpu.VMEM((N, HW, Cp), jnp.float32),
            pltpu.VMEM((N, HW, Cp), jnp.bfloat16),
            pltpu.VMEM((N, HW, Cp), jnp.bfloat16),
            pltpu.VMEM((2, Cp), jnp.float32),
            pltpu.VMEM((2, Cp), jnp.float32),
            pltpu.VMEM((2, Cp), jnp.float32),
            pltpu.VMEM((2, Cp), jnp.float32),
            pltpu.VMEM((2 * (H + 2) * W, 3 * Cp), jnp.bfloat16),
            pltpu.VMEM((HW, 5 * Cp), jnp.bfloat16),
        ],
        compiler_params=pltpu.CompilerParams(
            dimension_semantics=("arbitrary", "arbitrary"),
            vmem_limit_bytes=100 * 1024 * 1024),
    )(x0, gbrow(l0_g1), gbrow(l0_b1), _prep_conv_w(l0_w1, Cp, Cp),
      gbrow(l0_g2), gbrow(l0_b2), _prep_conv_w(l0_w2, Cp, Cp),
      gbrow(l1_g1), gbrow(l1_b1), _prep_conv_w(l1_w1, Cp, Cp),
      gbrow(l1_g2), gbrow(l1_b2), wqa, wqb, wqc, wqd,
      gbrow(l1_g3), gbrow(l1_b3))

    out = out.reshape(N, Cop, 2 * H, 2 * W)
    if Cop != Co:
        out = out[:, :Co]
    return out
